# baseline (speedup 1.0000x reference)
import functools

import jax
import jax.numpy as jnp
from jax import lax
from jax.experimental import pallas as pl
from jax.experimental.pallas import tpu as pltpu
from jax.experimental.pallas import tpu_sc as plsc

_B = 4
_S = 2048
_D = 1024
_NC = 2
_NS = 16
_NW = _NC * _NS
_S_PER_W = _S // _NW
_R = 16
_CH = _S_PER_W // _R
_NU = _CH * _B
_LANES = 16
_NI = 5


def _sc_body(x_hbm, tab_hbm, out_hbm, *scr):
    ibufs = scr[:_NI]
    tbufs = scr[_NI:_NI + 2]
    in_sems = scr[_NI + 2:2 * _NI + 2]
    out_sems = scr[2 * _NI + 2:3 * _NI + 2]
    tab_sems = scr[3 * _NI + 2:]

    wid = lax.axis_index("s") * _NC + lax.axis_index("c")
    base = wid * _S_PER_W

    def in_copy(u):
        c, b = u // _B, u % _B
        row = base + c * _R
        return pltpu.make_async_copy(
            x_hbm.at[b, pl.ds(row, _R), :], ibufs[u % _NI], in_sems[u % _NI])

    def out_copy(u):
        c, b = u // _B, u % _B
        row = base + c * _R
        return pltpu.make_async_copy(
            ibufs[u % _NI], out_hbm.at[b, pl.ds(row, _R), :], out_sems[u % _NI])

    def tab_copy(c):
        return pltpu.make_async_copy(
            tab_hbm.at[pl.ds(base + c * _R, _R), :], tbufs[c % 2], tab_sems[c % 2])

    for u in range(min(_NI, _NU)):
        in_copy(u).start()
    for c in range(min(2, _CH)):
        tab_copy(c).start()

    for u in range(_NU):
        c = u // _B
        in_copy(u).wait()
        if u % _B == 0:
            tab_copy(c).wait()

        ibuf, tbuf = ibufs[u % _NI], tbufs[c % 2]

        def _add_blk(m, carry):
            r = m // 4
            jb = (m % 4) * (_D // 4)
            for k in range(_D // 4 // _LANES):
                sl = pl.ds(jb + k * _LANES, _LANES)
                plsc.addupdate(ibuf.at[r, sl], tbuf[r, sl])
            return carry

        lax.fori_loop(0, _R * 4, _add_blk, 0)
        out_copy(u).start()
        v = u + _NI - 1
        if v < _NU and u >= 1:
            out_copy(u - 1).wait()
            in_copy(v).start()
        if u % _B == _B - 1 and c + 2 < _CH:
            tab_copy(c + 2).start()

    for u in range(max(0, _NU - _NI), _NU):
        out_copy(u).wait()


@functools.partial(jax.jit)
def _sc_add(x, tab):
    kern = pl.kernel(
        _sc_body,
        out_type=jax.ShapeDtypeStruct((_B, _S, _D), jnp.float32),
        mesh=plsc.VectorSubcoreMesh(core_axis_name="c", subcore_axis_name="s"),
        scratch_types=(
            [pltpu.VMEM((_R, _D), jnp.float32) for _ in range(_NI + 2)]
            + [pltpu.SemaphoreType.DMA for _ in range(2 * _NI + 2)]
        ),
    )
    return kern(x, tab)


def kernel(x, position_embedding):
    batch, seq_len, d = x.shape
    return _sc_add(x, position_embedding[:seq_len])


# speedup vs baseline: 1.4820x; 1.4820x over previous
import functools

import jax
import jax.numpy as jnp
from jax import lax
from jax.experimental import pallas as pl
from jax.experimental.pallas import tpu as pltpu
from jax.experimental.pallas import tpu_sc as plsc

_B = 4
_S = 2048
_D = 1024
_NC = 2
_NS = 16
_NW = _NC * _NS
_S_PER_W = _S // _NW
_R = 16
_CH = _S_PER_W // _R
_NU = _CH * _B
_LANES = 16
_NI = 3
_NO = 3


def _sc_body(x_hbm, tab_hbm, out_hbm, *scr):
    ibufs = scr[:_NI]
    obufs = scr[_NI:_NI + _NO]
    tbufs = scr[_NI + _NO:_NI + _NO + 2]
    in_sems = scr[_NI + _NO + 2:2 * _NI + _NO + 2]
    out_sems = scr[2 * _NI + _NO + 2:2 * _NI + 2 * _NO + 2]
    tab_sems = scr[2 * _NI + 2 * _NO + 2:]

    wid = lax.axis_index("s") * _NC + lax.axis_index("c")
    base = wid * _S_PER_W

    def in_copy(u):
        c, b = u // _B, u % _B
        row = base + c * _R
        return pltpu.make_async_copy(
            x_hbm.at[b, pl.ds(row, _R), :], ibufs[u % _NI], in_sems[u % _NI])

    def out_copy(u):
        c, b = u // _B, u % _B
        row = base + c * _R
        return pltpu.make_async_copy(
            obufs[u % _NO], out_hbm.at[b, pl.ds(row, _R), :], out_sems[u % _NO])

    def tab_copy(c):
        return pltpu.make_async_copy(
            tab_hbm.at[pl.ds(base + c * _R, _R), :], tbufs[c % 2], tab_sems[c % 2])

    for u in range(min(_NI, _NU)):
        in_copy(u).start()
    for c in range(min(2, _CH)):
        tab_copy(c).start()

    for u in range(_NU):
        c = u // _B
        in_copy(u).wait()
        if u % _B == 0:
            tab_copy(c).wait()
        if u - _NO >= 0:
            out_copy(u - _NO).wait()
        if u + _NI - 1 < _NU and u >= 1:
            in_copy(u + _NI - 1).start()

        ibuf, obuf, tbuf = ibufs[u % _NI], obufs[u % _NO], tbufs[c % 2]

        def _add_row(r, carry):
            for j in range(_D // _LANES):
                sl = pl.ds(j * _LANES, _LANES)
                obuf[r, sl] = ibuf[r, sl] + tbuf[r, sl]
            return carry

        lax.fori_loop(0, _R, _add_row, 0)
        out_copy(u).start()
        if u % _B == _B - 1 and c + 2 < _CH:
            tab_copy(c + 2).start()

    for u in range(max(0, _NU - _NO), _NU):
        out_copy(u).wait()


@functools.partial(jax.jit)
def _sc_add(x, tab):
    kern = pl.kernel(
        _sc_body,
        out_type=jax.ShapeDtypeStruct((_B, _S, _D), jnp.float32),
        mesh=plsc.VectorSubcoreMesh(core_axis_name="c", subcore_axis_name="s"),
        scratch_types=(
            [pltpu.VMEM((_R, _D), jnp.float32) for _ in range(_NI + _NO + 2)]
            + [pltpu.SemaphoreType.DMA for _ in range(_NI + _NO + 2)]
        ),
    )
    return kern(x, tab)


def kernel(x, position_embedding):
    batch, seq_len, d = x.shape
    return _sc_add(x, position_embedding[:seq_len])
